# baseline (speedup 1.0000x reference)
import jax
import jax.numpy as jnp
from jax import lax
from jax.experimental import pallas as pl
from jax.experimental.pallas import tpu as pltpu

_EPS = 1e-06
_D = 1024
_NMOD = 64
_SEQ = 8192
_BLK = 256
_NBLK = _SEQ // _BLK
_WAYS = 8
_NB = _NBLK // _WAYS


def _fused_body(*refs):
    x_refs = refs[:_WAYS]
    keep_refs = refs[_WAYS:2 * _WAYS]
    w0_ref, w1_ref, b_ref = refs[2 * _WAYS:2 * _WAYS + 3]
    l0_ref, l1_ref, s0_ref, s1_ref = refs[2 * _WAYS + 3:2 * _WAYS + 7]
    acc_ref, cnt_ref = refs[2 * _WAYS + 7:]

    k = pl.program_id(1)

    part = jnp.zeros((8, _D), jnp.float32)
    csum = jnp.float32(0.0)
    for i in range(_WAYS):
        keep = keep_refs[i][0]
        xm = x_refs[i][0] * keep.reshape(_BLK, 1)
        part = part + jnp.sum(xm.reshape(_BLK // 8, 8, _D), axis=0)
        csum = csum + jnp.sum(keep)

    @pl.when(k == 0)
    def _init():
        acc_ref[...] = part
        cnt_ref[0] = csum

    @pl.when(k > 0)
    def _accum():
        acc_ref[...] += part
        cnt_ref[0] += csum

    @pl.when(k == _NB - 1)
    def _finish():
        total = jnp.sum(acc_ref[...], axis=0, keepdims=True)
        feats = total / (cnt_ref[0] + _EPS)
        dn = (((1,), (1,)), ((), ()))
        l0 = lax.dot_general(feats, w0_ref[...], dn,
                             preferred_element_type=jnp.float32) + b_ref[0, :_NMOD]
        l1 = lax.dot_general(feats, w1_ref[...], dn,
                             preferred_element_type=jnp.float32) + b_ref[0, _NMOD:]
        l0_ref[0] = l0
        l1_ref[0] = l1
        iota = lax.broadcasted_iota(jnp.int32, (1, _NMOD), 1)
        m0 = jnp.max(l0, axis=1, keepdims=True)
        m1 = jnp.max(l1, axis=1, keepdims=True)
        s0_ref[0] = jnp.min(jnp.where(l0 >= m0, iota, _NMOD), axis=1,
                            keepdims=True)
        s1_ref[0] = jnp.min(jnp.where(l1 >= m1, iota, _NMOD), axis=1,
                            keepdims=True)


def _x_spec(i):
    return pl.BlockSpec((1, _BLK, _D), lambda b_, k, i=i: (b_, i * _NB + k, 0))


def _keep_spec(i):
    return pl.BlockSpec((1, 1, _BLK),
                        lambda b_, k, i=i: (b_ * _NBLK + i * _NB + k, 0, 0))


@jax.jit
def _fused(x, keep, w0, w1, b):
    bsz = x.shape[0]
    grid = (bsz, _NB)
    out = pl.pallas_call(
        _fused_body,
        grid=grid,
        in_specs=(
            [_x_spec(i) for i in range(_WAYS)]
            + [_keep_spec(i) for i in range(_WAYS)]
            + [
                pl.BlockSpec((_NMOD, _D), lambda b_, k: (0, 0)),
                pl.BlockSpec((_NMOD, _D), lambda b_, k: (0, 0)),
                pl.BlockSpec((1, 2 * _NMOD), lambda b_, k: (0, 0)),
            ]
        ),
        out_specs=[
            pl.BlockSpec((1, 1, _NMOD), lambda b_, k: (b_, 0, 0)),
            pl.BlockSpec((1, 1, _NMOD), lambda b_, k: (b_, 0, 0)),
            pl.BlockSpec((1, 1, 1), lambda b_, k: (b_, 0, 0)),
            pl.BlockSpec((1, 1, 1), lambda b_, k: (b_, 0, 0)),
        ],
        out_shape=[
            jax.ShapeDtypeStruct((bsz, 1, _NMOD), jnp.float32),
            jax.ShapeDtypeStruct((bsz, 1, _NMOD), jnp.float32),
            jax.ShapeDtypeStruct((bsz, 1, 1), jnp.int32),
            jax.ShapeDtypeStruct((bsz, 1, 1), jnp.int32),
        ],
        scratch_shapes=[
            pltpu.VMEM((8, _D), jnp.float32),
            pltpu.SMEM((1,), jnp.float32),
        ],
    )(*([x] * _WAYS + [keep] * _WAYS + [w0, w1, b]))
    return out


def kernel(x, padding_mask, W_out, b_out):
    bsz = x.shape[0]
    x = x.reshape(bsz, _SEQ, _D)
    keep = 1.0 - padding_mask.reshape(bsz * _NBLK, 1, _BLK).astype(jnp.float32)
    w0 = W_out[:_NMOD]
    w1 = W_out[_NMOD:]
    b = b_out.reshape(1, 2 * _NMOD)
    l0, l1, s0, s1 = _fused(x, keep, w0, w1, b)
    logits = jnp.concatenate([l0, l1], axis=1)
    selection = jnp.concatenate([s0[:, :, 0], s1[:, :, 0]], axis=1)
    return (logits, selection, selection)


# speedup vs baseline: 1.0001x; 1.0001x over previous
import jax
import jax.numpy as jnp
from jax import lax
from jax.experimental import pallas as pl
from jax.experimental.pallas import tpu as pltpu

_EPS = 1e-06
_D = 1024
_NMOD = 64
_SEQ = 8192
_BLK = 512
_NBLK = _SEQ // _BLK
_WAYS = 8
_NB = _NBLK // _WAYS


def _fused_body(*refs):
    x_refs = refs[:_WAYS]
    keep_refs = refs[_WAYS:2 * _WAYS]
    w0_ref, w1_ref, b_ref = refs[2 * _WAYS:2 * _WAYS + 3]
    l0_ref, l1_ref, s0_ref, s1_ref = refs[2 * _WAYS + 3:2 * _WAYS + 7]
    acc_ref, cnt_ref = refs[2 * _WAYS + 7:]

    k = pl.program_id(1)

    part = jnp.zeros((8, _D), jnp.float32)
    csum = jnp.float32(0.0)
    for i in range(_WAYS):
        keep = keep_refs[i][0]
        xm = x_refs[i][0] * keep.reshape(_BLK, 1)
        part = part + jnp.sum(xm.reshape(_BLK // 8, 8, _D), axis=0)
        csum = csum + jnp.sum(keep)

    @pl.when(k == 0)
    def _init():
        acc_ref[...] = part
        cnt_ref[0] = csum

    @pl.when(k > 0)
    def _accum():
        acc_ref[...] += part
        cnt_ref[0] += csum

    @pl.when(k == _NB - 1)
    def _finish():
        total = jnp.sum(acc_ref[...], axis=0, keepdims=True)
        feats = total / (cnt_ref[0] + _EPS)
        dn = (((1,), (1,)), ((), ()))
        l0 = lax.dot_general(feats, w0_ref[...], dn,
                             preferred_element_type=jnp.float32) + b_ref[0, :_NMOD]
        l1 = lax.dot_general(feats, w1_ref[...], dn,
                             preferred_element_type=jnp.float32) + b_ref[0, _NMOD:]
        l0_ref[0] = l0
        l1_ref[0] = l1
        iota = lax.broadcasted_iota(jnp.int32, (1, _NMOD), 1)
        m0 = jnp.max(l0, axis=1, keepdims=True)
        m1 = jnp.max(l1, axis=1, keepdims=True)
        s0_ref[0] = jnp.min(jnp.where(l0 >= m0, iota, _NMOD), axis=1,
                            keepdims=True)
        s1_ref[0] = jnp.min(jnp.where(l1 >= m1, iota, _NMOD), axis=1,
                            keepdims=True)


def _x_spec(i):
    return pl.BlockSpec((1, _BLK, _D), lambda b_, k, i=i: (b_, i * _NB + k, 0))


def _keep_spec(i):
    return pl.BlockSpec((1, 1, _BLK),
                        lambda b_, k, i=i: (b_ * _NBLK + i * _NB + k, 0, 0))


@jax.jit
def _fused(x, keep, w0, w1, b):
    bsz = x.shape[0]
    grid = (bsz, _NB)
    out = pl.pallas_call(
        _fused_body,
        grid=grid,
        in_specs=(
            [_x_spec(i) for i in range(_WAYS)]
            + [_keep_spec(i) for i in range(_WAYS)]
            + [
                pl.BlockSpec((_NMOD, _D), lambda b_, k: (0, 0)),
                pl.BlockSpec((_NMOD, _D), lambda b_, k: (0, 0)),
                pl.BlockSpec((1, 2 * _NMOD), lambda b_, k: (0, 0)),
            ]
        ),
        out_specs=[
            pl.BlockSpec((1, 1, _NMOD), lambda b_, k: (b_, 0, 0)),
            pl.BlockSpec((1, 1, _NMOD), lambda b_, k: (b_, 0, 0)),
            pl.BlockSpec((1, 1, 1), lambda b_, k: (b_, 0, 0)),
            pl.BlockSpec((1, 1, 1), lambda b_, k: (b_, 0, 0)),
        ],
        out_shape=[
            jax.ShapeDtypeStruct((bsz, 1, _NMOD), jnp.float32),
            jax.ShapeDtypeStruct((bsz, 1, _NMOD), jnp.float32),
            jax.ShapeDtypeStruct((bsz, 1, 1), jnp.int32),
            jax.ShapeDtypeStruct((bsz, 1, 1), jnp.int32),
        ],
        scratch_shapes=[
            pltpu.VMEM((8, _D), jnp.float32),
            pltpu.SMEM((1,), jnp.float32),
        ],
    )(*([x] * _WAYS + [keep] * _WAYS + [w0, w1, b]))
    return out


def kernel(x, padding_mask, W_out, b_out):
    bsz = x.shape[0]
    x = x.reshape(bsz, _SEQ, _D)
    keep = 1.0 - padding_mask.reshape(bsz * _NBLK, 1, _BLK).astype(jnp.float32)
    w0 = W_out[:_NMOD]
    w1 = W_out[_NMOD:]
    b = b_out.reshape(1, 2 * _NMOD)
    l0, l1, s0, s1 = _fused(x, keep, w0, w1, b)
    logits = jnp.concatenate([l0, l1], axis=1)
    selection = jnp.concatenate([s0[:, :, 0], s1[:, :, 0]], axis=1)
    return (logits, selection, selection)


# speedup vs baseline: 1.0532x; 1.0531x over previous
import jax
import jax.numpy as jnp
from jax import lax
from jax.experimental import pallas as pl
from jax.experimental.pallas import tpu as pltpu

_EPS = 1e-06
_D = 1024
_NMOD = 64
_SEQ = 8192
_BLK = 256
_NBLK = _SEQ // _BLK
_WAYS = 8
_NB = _NBLK // _WAYS


def _fused_body(*refs):
    x_refs = refs[:_WAYS]
    keep_refs = refs[_WAYS:2 * _WAYS]
    w0_ref, w1_ref, b_ref = refs[2 * _WAYS:2 * _WAYS + 3]
    l0_ref, l1_ref, s0_ref, s1_ref = refs[2 * _WAYS + 3:2 * _WAYS + 7]
    acc_ref, cnt_ref = refs[2 * _WAYS + 7:]

    k = pl.program_id(1)

    part = jnp.zeros((8, _D), jnp.float32)
    csum = jnp.float32(0.0)
    for i in range(_WAYS):
        keep = keep_refs[i][0]
        xm = x_refs[i][0] * keep.reshape(_BLK, 1)
        part = part + jnp.sum(xm.reshape(_BLK // 8, 8, _D), axis=0)
        csum = csum + jnp.sum(keep)

    @pl.when(k == 0)
    def _init():
        acc_ref[...] = part
        cnt_ref[0] = csum

    @pl.when(k > 0)
    def _accum():
        acc_ref[...] += part
        cnt_ref[0] += csum

    @pl.when(k == _NB - 1)
    def _finish():
        total = jnp.sum(acc_ref[...], axis=0, keepdims=True)
        feats = total / (cnt_ref[0] + _EPS)
        dn = (((1,), (1,)), ((), ()))
        l0 = lax.dot_general(feats, w0_ref[...], dn,
                             preferred_element_type=jnp.float32) + b_ref[0, :_NMOD]
        l1 = lax.dot_general(feats, w1_ref[...], dn,
                             preferred_element_type=jnp.float32) + b_ref[0, _NMOD:]
        l0_ref[0] = l0
        l1_ref[0] = l1
        iota = lax.broadcasted_iota(jnp.int32, (1, _NMOD), 1)
        m0 = jnp.max(l0, axis=1, keepdims=True)
        m1 = jnp.max(l1, axis=1, keepdims=True)
        s0_ref[0] = jnp.min(jnp.where(l0 >= m0, iota, _NMOD), axis=1,
                            keepdims=True)
        s1_ref[0] = jnp.min(jnp.where(l1 >= m1, iota, _NMOD), axis=1,
                            keepdims=True)


def _x_spec(i):
    return pl.BlockSpec((1, _BLK, _D), lambda b_, k, i=i: (b_, i * _NB + k, 0))


def _keep_spec(i):
    return pl.BlockSpec((1, 1, _BLK),
                        lambda b_, k, i=i: (b_ * _NBLK + i * _NB + k, 0, 0))


@jax.jit
def _fused(x, keep, w0, w1, b):
    bsz = x.shape[0]
    grid = (bsz, _NB)
    out = pl.pallas_call(
        _fused_body,
        grid=grid,
        in_specs=(
            [_x_spec(i) for i in range(_WAYS)]
            + [_keep_spec(i) for i in range(_WAYS)]
            + [
                pl.BlockSpec((_NMOD, _D), lambda b_, k: (0, 0)),
                pl.BlockSpec((_NMOD, _D), lambda b_, k: (0, 0)),
                pl.BlockSpec((1, 2 * _NMOD), lambda b_, k: (0, 0)),
            ]
        ),
        out_specs=[
            pl.BlockSpec((1, 1, _NMOD), lambda b_, k: (b_, 0, 0)),
            pl.BlockSpec((1, 1, _NMOD), lambda b_, k: (b_, 0, 0)),
            pl.BlockSpec((1, 1, 1), lambda b_, k: (b_, 0, 0)),
            pl.BlockSpec((1, 1, 1), lambda b_, k: (b_, 0, 0)),
        ],
        out_shape=[
            jax.ShapeDtypeStruct((bsz, 1, _NMOD), jnp.float32),
            jax.ShapeDtypeStruct((bsz, 1, _NMOD), jnp.float32),
            jax.ShapeDtypeStruct((bsz, 1, 1), jnp.int32),
            jax.ShapeDtypeStruct((bsz, 1, 1), jnp.int32),
        ],
        scratch_shapes=[
            pltpu.VMEM((8, _D), jnp.float32),
            pltpu.SMEM((1,), jnp.float32),
        ],
    )(*([x] * _WAYS + [keep] * _WAYS + [w0, w1, b]))
    return out


def kernel(x, padding_mask, W_out, b_out):
    bsz = x.shape[0]
    x = x.reshape(bsz, _SEQ, _D)
    keep = 1.0 - padding_mask.reshape(bsz * _NBLK, 1, _BLK).astype(jnp.float32)
    w0 = W_out[:_NMOD]
    w1 = W_out[_NMOD:]
    b = b_out.reshape(1, 2 * _NMOD)
    l0, l1, s0, s1 = _fused(x, keep, w0, w1, b)
    logits = jnp.concatenate([l0, l1], axis=1)
    selection = jnp.concatenate([s0[:, :, 0], s1[:, :, 0]], axis=1)
    return (logits, selection, selection)
